# Initial kernel scaffold; baseline (speedup 1.0000x reference)
import jax
import jax.numpy as jnp
from jax.experimental import pallas as pl


def kernel(xyz, feat, params):
    raise NotImplementedError("write your pallas kernel here")


# speedup vs baseline: 10.2423x; 10.2423x over previous
import functools

import jax
import jax.numpy as jnp
from jax import lax
from jax.experimental import pallas as pl
from jax.experimental.pallas import tpu as pltpu
from jax.experimental.pallas import tpu_sc as plsc

KNN = 20
EPS = 1e-5


def _lrelu(z):
    return jnp.maximum(z, 0.2 * z)


def _bn_mv(s_ref, q_ref, count):
    mean = s_ref[...] / count
    var = q_ref[...] / count - mean * mean
    return mean, jnp.sqrt(var + EPS)


def _bn_apply(h, mean, sq, g_ref, b_ref):
    return g_ref[...] * (h - mean) / sq + b_ref[...]


def _knn_body(rows_ref, x_ref, xxr_ref, xxc_ref, out_ref):
    rows = rows_ref[0]
    xf = x_ref[0]
    e = jnp.dot(rows, xf, preferred_element_type=jnp.float32)
    score = ((-xxr_ref[0]) + 2.0 * e) - xxc_ref[0]
    r, n = score.shape
    iota = lax.broadcasted_iota(jnp.int32, (r, n), 1)
    cols = []
    for _ in range(KNN):
        m = jnp.max(score, axis=1, keepdims=True)
        cand = jnp.where(score == m, iota, n)
        j = jnp.min(cand, axis=1, keepdims=True)
        cols.append(j)
        score = jnp.where(iota == j, -jnp.inf, score)
    out_ref[0] = jnp.concatenate(cols, axis=1)


def _knn(xt, x, xx, r=256):
    b, c, n = x.shape
    return pl.pallas_call(
        _knn_body,
        grid=(b, n // r),
        in_specs=[
            pl.BlockSpec((1, r, c), lambda i, j: (i, j, 0)),
            pl.BlockSpec((1, c, n), lambda i, j: (i, 0, 0)),
            pl.BlockSpec((1, r, 1), lambda i, j: (i, j, 0)),
            pl.BlockSpec((1, 1, n), lambda i, j: (i, 0, 0)),
        ],
        out_specs=pl.BlockSpec((1, r, KNN), lambda i, j: (i, j, 0)),
        out_shape=jax.ShapeDtypeStruct((b, n, KNN), jnp.int32),
    )(xt, x, xx[:, :, None], xx[:, None, :])


def _gather(table, idxf):
    m_total = idxf.shape[0]
    d = table.shape[1]
    info = plsc.get_sparse_core_info()
    nw = info.num_cores * info.num_subcores
    per_w = m_total // nw
    ch = 128
    nch = per_w // ch
    mesh = plsc.VectorSubcoreMesh(core_axis_name="c", subcore_axis_name="s")

    @functools.partial(
        pl.kernel,
        out_type=jax.ShapeDtypeStruct((m_total, d), jnp.float32),
        mesh=mesh,
        compiler_params=pltpu.CompilerParams(use_tc_tiling_on_sc=False),
        scratch_types=[
            pltpu.VMEM((per_w,), jnp.int32),
            pltpu.VMEM((ch, d), jnp.float32),
            pltpu.SemaphoreType.DMA,
        ],
    )
    def k(table_hbm, idx_hbm, out_hbm, idx_v, rows_v, sem):
        wid = lax.axis_index("s") * info.num_cores + lax.axis_index("c")
        base = wid * per_w
        pltpu.sync_copy(idx_hbm.at[pl.ds(base, per_w)], idx_v)

        def body(c, carry):
            pltpu.async_copy(
                table_hbm.at[idx_v.at[pl.ds(c * ch, ch)]], rows_v, sem
            ).wait()
            pltpu.sync_copy(rows_v, out_hbm.at[pl.ds(base + c * ch, ch)])
            return carry

        lax.fori_loop(0, nch, body, 0)

    return k(table, idxf)


def _estats_body(g_ref, xt_ref, w_ref, s_ref, q_ref, sacc, qacc):
    step = pl.program_id(0) * pl.num_programs(1) + pl.program_id(1)
    nstep = pl.num_programs(0) * pl.num_programs(1)

    @pl.when(step == 0)
    def _():
        sacc[...] = jnp.zeros_like(sacc)
        qacc[...] = jnp.zeros_like(qacc)

    xt = xt_ref[0]
    hs = None
    hq = None
    for k in range(KNN):
        hcat = jnp.concatenate([g_ref[0, k], xt], axis=1)
        h = jnp.dot(hcat, w_ref[...], preferred_element_type=jnp.float32)
        hs = h if hs is None else hs + h
        hq = h * h if hq is None else hq + h * h
    sacc[...] += jnp.sum(hs, axis=0, keepdims=True)
    qacc[...] += jnp.sum(hq, axis=0, keepdims=True)

    @pl.when(step == nstep - 1)
    def _():
        s_ref[...] = sacc[...]
        q_ref[...] = qacc[...]


def _estats(g4, xt, w, tn=512):
    b, _, n, dp = g4.shape
    co = w.shape[1]
    return pl.pallas_call(
        _estats_body,
        grid=(b, n // tn),
        in_specs=[
            pl.BlockSpec((1, KNN, tn, dp), lambda i, j: (i, 0, j, 0)),
            pl.BlockSpec((1, tn, xt.shape[2]), lambda i, j: (i, j, 0)),
            pl.BlockSpec(w.shape, lambda i, j: (0, 0)),
        ],
        out_specs=[
            pl.BlockSpec((1, co), lambda i, j: (0, 0)),
            pl.BlockSpec((1, co), lambda i, j: (0, 0)),
        ],
        out_shape=[
            jax.ShapeDtypeStruct((1, co), jnp.float32),
            jax.ShapeDtypeStruct((1, co), jnp.float32),
        ],
        scratch_shapes=[
            pltpu.VMEM((1, co), jnp.float32),
            pltpu.VMEM((1, co), jnp.float32),
        ],
    )(g4, xt, w)


def _econv_body(g_ref, xt_ref, w_ref, w2_ref, s1_ref, q1_ref, g1_ref,
                b1_ref, m_ref, s_ref, q_ref, sacc, qacc, *, count1):
    step = pl.program_id(0) * pl.num_programs(1) + pl.program_id(1)
    nstep = pl.num_programs(0) * pl.num_programs(1)

    @pl.when(step == 0)
    def _():
        sacc[...] = jnp.zeros_like(sacc)
        qacc[...] = jnp.zeros_like(qacc)

    m1, sq1 = _bn_mv(s1_ref, q1_ref, count1)
    xt = xt_ref[0]
    mx = None
    hs = None
    hq = None
    for k in range(KNN):
        hcat = jnp.concatenate([g_ref[0, k], xt], axis=1)
        h = jnp.dot(hcat, w_ref[...], preferred_element_type=jnp.float32)
        a = _lrelu(_bn_apply(h, m1, sq1, g1_ref, b1_ref))
        h2 = jnp.dot(a, w2_ref[...], preferred_element_type=jnp.float32)
        mx = h2 if mx is None else jnp.maximum(mx, h2)
        hs = h2 if hs is None else hs + h2
        hq = h2 * h2 if hq is None else hq + h2 * h2
    m_ref[0] = mx
    sacc[...] += jnp.sum(hs, axis=0, keepdims=True)
    qacc[...] += jnp.sum(hq, axis=0, keepdims=True)

    @pl.when(step == nstep - 1)
    def _():
        s_ref[...] = sacc[...]
        q_ref[...] = qacc[...]


def _econv(g4, xt, w, w2, s1, q1, g1, b1, count1, tn=512):
    b, _, n, dp = g4.shape
    co = w2.shape[1]
    return pl.pallas_call(
        functools.partial(_econv_body, count1=count1),
        grid=(b, n // tn),
        in_specs=[
            pl.BlockSpec((1, KNN, tn, dp), lambda i, j: (i, 0, j, 0)),
            pl.BlockSpec((1, tn, xt.shape[2]), lambda i, j: (i, j, 0)),
            pl.BlockSpec(w.shape, lambda i, j: (0, 0)),
            pl.BlockSpec(w2.shape, lambda i, j: (0, 0)),
            pl.BlockSpec(s1.shape, lambda i, j: (0, 0)),
            pl.BlockSpec(q1.shape, lambda i, j: (0, 0)),
            pl.BlockSpec(g1.shape, lambda i, j: (0, 0)),
            pl.BlockSpec(b1.shape, lambda i, j: (0, 0)),
        ],
        out_specs=[
            pl.BlockSpec((1, tn, co), lambda i, j: (i, j, 0)),
            pl.BlockSpec((1, co), lambda i, j: (0, 0)),
            pl.BlockSpec((1, co), lambda i, j: (0, 0)),
        ],
        out_shape=[
            jax.ShapeDtypeStruct((b, n, co), jnp.float32),
            jax.ShapeDtypeStruct((1, co), jnp.float32),
            jax.ShapeDtypeStruct((1, co), jnp.float32),
        ],
        scratch_shapes=[
            pltpu.VMEM((1, co), jnp.float32),
            pltpu.VMEM((1, co), jnp.float32),
        ],
    )(g4, xt, w, w2, s1, q1, g1, b1)


def _emax_body(g_ref, xt_ref, w_ref, s_ref, q_ref, g_bn_ref, b_bn_ref,
               o_ref, *, count):
    m, sq = _bn_mv(s_ref, q_ref, count)
    xt = xt_ref[0]
    mx = None
    for k in range(KNN):
        hcat = jnp.concatenate([g_ref[0, k], xt], axis=1)
        h = jnp.dot(hcat, w_ref[...], preferred_element_type=jnp.float32)
        mx = h if mx is None else jnp.maximum(mx, h)
    o_ref[0] = _lrelu(_bn_apply(mx, m, sq, g_bn_ref, b_bn_ref))


def _emax(g4, xt, w, s, q, g_bn, b_bn, count, tn=512):
    b, _, n, dp = g4.shape
    co = w.shape[1]
    return pl.pallas_call(
        functools.partial(_emax_body, count=count),
        grid=(b, n // tn),
        in_specs=[
            pl.BlockSpec((1, KNN, tn, dp), lambda i, j: (i, 0, j, 0)),
            pl.BlockSpec((1, tn, xt.shape[2]), lambda i, j: (i, j, 0)),
            pl.BlockSpec(w.shape, lambda i, j: (0, 0)),
            pl.BlockSpec(s.shape, lambda i, j: (0, 0)),
            pl.BlockSpec(q.shape, lambda i, j: (0, 0)),
            pl.BlockSpec(g_bn.shape, lambda i, j: (0, 0)),
            pl.BlockSpec(b_bn.shape, lambda i, j: (0, 0)),
        ],
        out_specs=pl.BlockSpec((1, tn, co), lambda i, j: (i, j, 0)),
        out_shape=jax.ShapeDtypeStruct((b, n, co), jnp.float32),
    )(g4, xt, w, s, q, g_bn, b_bn)


def _bnact_body(m_ref, s_ref, q_ref, g_ref, b_ref, o_ref, *, count):
    m, sq = _bn_mv(s_ref, q_ref, count)
    o_ref[0] = _lrelu(_bn_apply(m_ref[0], m, sq, g_ref, b_ref))


def _bnact(m, s, q, g_bn, b_bn, count, tn=512):
    b, n, co = m.shape
    return pl.pallas_call(
        functools.partial(_bnact_body, count=count),
        grid=(b, n // tn),
        in_specs=[
            pl.BlockSpec((1, tn, co), lambda i, j: (i, j, 0)),
            pl.BlockSpec(s.shape, lambda i, j: (0, 0)),
            pl.BlockSpec(q.shape, lambda i, j: (0, 0)),
            pl.BlockSpec(g_bn.shape, lambda i, j: (0, 0)),
            pl.BlockSpec(b_bn.shape, lambda i, j: (0, 0)),
        ],
        out_specs=pl.BlockSpec((1, tn, co), lambda i, j: (i, j, 0)),
        out_shape=jax.ShapeDtypeStruct((b, n, co), jnp.float32),
    )(m, s, q, g_bn, b_bn)


def _t1_body(h_ref, w6_ref, s_ref, q_ref, umax_ref, sacc, qacc):
    nt = pl.program_id(1)
    step = pl.program_id(0) * pl.num_programs(1) + nt
    nstep = pl.num_programs(0) * pl.num_programs(1)

    @pl.when(step == 0)
    def _():
        sacc[...] = jnp.zeros_like(sacc)
        qacc[...] = jnp.zeros_like(qacc)

    u = jnp.dot(h_ref[0], w6_ref[...], preferred_element_type=jnp.float32)
    rmax = jnp.max(u, axis=0, keepdims=True)

    @pl.when(nt == 0)
    def _():
        umax_ref[0] = rmax

    @pl.when(nt != 0)
    def _():
        umax_ref[0] = jnp.maximum(umax_ref[0], rmax)

    sacc[...] += jnp.sum(u, axis=0, keepdims=True)
    qacc[...] += jnp.sum(u * u, axis=0, keepdims=True)

    @pl.when(step == nstep - 1)
    def _():
        s_ref[...] = sacc[...]
        q_ref[...] = qacc[...]


def _t1(h, w6, tn=512):
    b, n, ci = h.shape
    co = w6.shape[1]
    return pl.pallas_call(
        _t1_body,
        grid=(b, n // tn),
        in_specs=[
            pl.BlockSpec((1, tn, ci), lambda i, j: (i, j, 0)),
            pl.BlockSpec(w6.shape, lambda i, j: (0, 0)),
        ],
        out_specs=[
            pl.BlockSpec((1, co), lambda i, j: (0, 0)),
            pl.BlockSpec((1, co), lambda i, j: (0, 0)),
            pl.BlockSpec((1, 1, co), lambda i, j: (i, 0, 0)),
        ],
        out_shape=[
            jax.ShapeDtypeStruct((1, co), jnp.float32),
            jax.ShapeDtypeStruct((1, co), jnp.float32),
            jax.ShapeDtypeStruct((b, 1, co), jnp.float32),
        ],
        scratch_shapes=[
            pltpu.VMEM((1, co), jnp.float32),
            pltpu.VMEM((1, co), jnp.float32),
        ],
    )(h, w6)


def _tail_chain(h_ref, umax_ref, stats6, w7g_ref, w7x_ref, *, count):
    s6, q6, g6, b6 = stats6
    m6, sq6 = _bn_mv(s6, q6, count)
    gv = _lrelu(_bn_apply(umax_ref[0], m6, sq6, g6, b6))
    cb = jnp.dot(gv, w7g_ref[...], preferred_element_type=jnp.float32)
    return jnp.dot(h_ref[0], w7x_ref[...], preferred_element_type=jnp.float32) + cb


def _t2_body(h_ref, umax_ref, s6, q6, g6, b6, w7g_ref, w7x_ref, s_ref, q_ref,
             sacc, qacc, *, count):
    step = pl.program_id(0) * pl.num_programs(1) + pl.program_id(1)
    nstep = pl.num_programs(0) * pl.num_programs(1)

    @pl.when(step == 0)
    def _():
        sacc[...] = jnp.zeros_like(sacc)
        qacc[...] = jnp.zeros_like(qacc)

    v = _tail_chain(h_ref, umax_ref, (s6, q6, g6, b6), w7g_ref, w7x_ref, count=count)
    sacc[...] += jnp.sum(v, axis=0, keepdims=True)
    qacc[...] += jnp.sum(v * v, axis=0, keepdims=True)

    @pl.when(step == nstep - 1)
    def _():
        s_ref[...] = sacc[...]
        q_ref[...] = qacc[...]


def _t3_body(h_ref, umax_ref, s6, q6, g6, b6, w7g_ref, w7x_ref, s7, q7, g7, b7,
             w8_ref, s_ref, q_ref, sacc, qacc, *, count):
    step = pl.program_id(0) * pl.num_programs(1) + pl.program_id(1)
    nstep = pl.num_programs(0) * pl.num_programs(1)

    @pl.when(step == 0)
    def _():
        sacc[...] = jnp.zeros_like(sacc)
        qacc[...] = jnp.zeros_like(qacc)

    v = _tail_chain(h_ref, umax_ref, (s6, q6, g6, b6), w7g_ref, w7x_ref, count=count)
    m7, sq7 = _bn_mv(s7, q7, count)
    w = jnp.dot(_lrelu(_bn_apply(v, m7, sq7, g7, b7)), w8_ref[...],
                preferred_element_type=jnp.float32)
    sacc[...] += jnp.sum(w, axis=0, keepdims=True)
    qacc[...] += jnp.sum(w * w, axis=0, keepdims=True)

    @pl.when(step == nstep - 1)
    def _():
        s_ref[...] = sacc[...]
        q_ref[...] = qacc[...]


def _t4_body(h_ref, umax_ref, s6, q6, g6, b6, w7g_ref, w7x_ref, s7, q7, g7, b7,
             w8_ref, s8, q8, g8, b8, w9_ref, o_ref, *, count):
    v = _tail_chain(h_ref, umax_ref, (s6, q6, g6, b6), w7g_ref, w7x_ref, count=count)
    m7, sq7 = _bn_mv(s7, q7, count)
    w = jnp.dot(_lrelu(_bn_apply(v, m7, sq7, g7, b7)), w8_ref[...],
                preferred_element_type=jnp.float32)
    m8, sq8 = _bn_mv(s8, q8, count)
    o_ref[0] = jnp.dot(_lrelu(_bn_apply(w, m8, sq8, g8, b8)), w9_ref[...],
                       preferred_element_type=jnp.float32)


def _small_specs(arrs):
    return [pl.BlockSpec(a.shape, lambda i, j: (0, 0)) for a in arrs]


def _t2(h, umax, stats6, w7g, w7x, count, tn=512):
    b, n, _ = h.shape
    co = w7x.shape[1]
    return pl.pallas_call(
        functools.partial(_t2_body, count=count),
        grid=(b, n // tn),
        in_specs=[
            pl.BlockSpec((1, tn, h.shape[2]), lambda i, j: (i, j, 0)),
            pl.BlockSpec((1, 1, umax.shape[2]), lambda i, j: (i, 0, 0)),
        ] + _small_specs(list(stats6) + [w7g, w7x]),
        out_specs=[
            pl.BlockSpec((1, co), lambda i, j: (0, 0)),
            pl.BlockSpec((1, co), lambda i, j: (0, 0)),
        ],
        out_shape=[
            jax.ShapeDtypeStruct((1, co), jnp.float32),
            jax.ShapeDtypeStruct((1, co), jnp.float32),
        ],
        scratch_shapes=[
            pltpu.VMEM((1, co), jnp.float32),
            pltpu.VMEM((1, co), jnp.float32),
        ],
    )(h, umax, *stats6, w7g, w7x)


def _t3(h, umax, stats6, w7g, w7x, stats7, w8, count, tn=512):
    b, n, _ = h.shape
    co = w8.shape[1]
    return pl.pallas_call(
        functools.partial(_t3_body, count=count),
        grid=(b, n // tn),
        in_specs=[
            pl.BlockSpec((1, tn, h.shape[2]), lambda i, j: (i, j, 0)),
            pl.BlockSpec((1, 1, umax.shape[2]), lambda i, j: (i, 0, 0)),
        ] + _small_specs(list(stats6) + [w7g, w7x] + list(stats7) + [w8]),
        out_specs=[
            pl.BlockSpec((1, co), lambda i, j: (0, 0)),
            pl.BlockSpec((1, co), lambda i, j: (0, 0)),
        ],
        out_shape=[
            jax.ShapeDtypeStruct((1, co), jnp.float32),
            jax.ShapeDtypeStruct((1, co), jnp.float32),
        ],
        scratch_shapes=[
            pltpu.VMEM((1, co), jnp.float32),
            pltpu.VMEM((1, co), jnp.float32),
        ],
    )(h, umax, *stats6, w7g, w7x, *stats7, w8)


def _t4(h, umax, stats6, w7g, w7x, stats7, w8, stats8, w9, count, tn=512):
    b, n, _ = h.shape
    co = w9.shape[1]
    return pl.pallas_call(
        functools.partial(_t4_body, count=count),
        grid=(b, n // tn),
        in_specs=[
            pl.BlockSpec((1, tn, h.shape[2]), lambda i, j: (i, j, 0)),
            pl.BlockSpec((1, 1, umax.shape[2]), lambda i, j: (i, 0, 0)),
        ] + _small_specs(list(stats6) + [w7g, w7x] + list(stats7) + [w8]
                         + list(stats8) + [w9]),
        out_specs=pl.BlockSpec((1, tn, co), lambda i, j: (i, j, 0)),
        out_shape=jax.ShapeDtypeStruct((b, n, co), jnp.float32),
    )(h, umax, *stats6, w7g, w7x, *stats7, w8, *stats8, w9)


def _edge_block(xt, x, xx, w, w2, bn1, bn2, b, n):
    cnt = float(b * n * KNN)
    idx = _knn(xt, x, xx)
    offs = (jnp.arange(b, dtype=jnp.int32) * n)[:, None, None]
    f = (jnp.transpose(idx, (0, 2, 1)) + offs).reshape(-1)
    g4 = _gather(xt.reshape(b * n, xt.shape[2]), f).reshape(b, KNN, n, xt.shape[2])
    s1, q1 = _estats(g4, xt, w)
    m2, s2, q2 = _econv(g4, xt, w, w2, s1, q1, bn1[0], bn1[1], cnt)
    return _bnact(m2, s2, q2, bn2[0], bn2[1], cnt)


def kernel(xyz, feat, params):
    p = params
    b, _, n = xyz.shape
    cnt_e = float(b * n * KNN)
    cnt_t = float(b * n)

    def col(v):
        return v.reshape(1, -1)

    x16 = jnp.concatenate(
        [xyz, feat, jnp.zeros((b, 10, n), jnp.float32)], axis=1)
    xt16 = jnp.transpose(x16, (0, 2, 1))
    w1 = p['W1']
    w1f = (jnp.zeros((32, 64), jnp.float32)
           .at[:6].set(w1[:, :6].T).at[16:22].set(w1[:, 6:].T))
    x6 = jnp.concatenate([xyz, feat], axis=1)
    xx1 = jnp.sum(x6 * x6, axis=1)
    x1t = _edge_block(xt16, x16, xx1, w1f, p['W2'].T,
                      (col(p['g1']), col(p['b1'])),
                      (col(p['g2']), col(p['b2'])), b, n)

    x1 = jnp.transpose(x1t, (0, 2, 1))
    x2t = _edge_block(x1t, x1, jnp.sum(x1 * x1, axis=1),
                      p['W3'].T, p['W4'].T,
                      (col(p['g3']), col(p['b3'])),
                      (col(p['g4']), col(p['b4'])), b, n)

    x2 = jnp.transpose(x2t, (0, 2, 1))
    idx3 = _knn(x2t, x2, jnp.sum(x2 * x2, axis=1))
    offs = (jnp.arange(b, dtype=jnp.int32) * n)[:, None, None]
    f3 = (jnp.transpose(idx3, (0, 2, 1)) + offs).reshape(-1)
    g43 = _gather(x2t.reshape(b * n, 64), f3).reshape(b, KNN, n, 64)
    s5, q5 = _estats(g43, x2t, p['W5'].T)
    x3t = _emax(g43, x2t, p['W5'].T, s5, q5, col(p['g5']), col(p['b5']), cnt_e)

    ht = jnp.concatenate([x1t, x2t, x3t], axis=2)
    s6, q6, umax = _t1(ht, p['W6'].T)
    stats6 = (s6, q6, col(p['g6']), col(p['b6']))
    w7g = p['W7'][:, :1024].T
    w7x = p['W7'][:, 1024:].T
    s7, q7 = _t2(ht, umax, stats6, w7g, w7x, cnt_t)
    stats7 = (s7, q7, col(p['g7']), col(p['b7']))
    s8, q8 = _t3(ht, umax, stats6, w7g, w7x, stats7, p['W8'].T, cnt_t)
    stats8 = (s8, q8, col(p['g8']), col(p['b8']))
    out_t = _t4(ht, umax, stats6, w7g, w7x, stats7, p['W8'].T, stats8,
                p['W9'].T, cnt_t)
    return jnp.transpose(out_t, (0, 2, 1))
